# baseline (speedup 1.0000x reference)
import functools

import jax
import jax.numpy as jnp
from jax.experimental import pallas as pl


def _silu(v):
    return v * jax.nn.sigmoid(v)


def _edge_body(pre01_ref, radial_ref, w1c_ref, b1_ref, w2_ref, b2_ref,
               cw1_ref, cb1_ref, cw2_ref, m_ref, phi_ref):
    pre1 = pre01_ref[...] + radial_ref[...] * w1c_ref[...] + b1_ref[...]
    t1 = _silu(pre1)
    z = jnp.dot(t1, w2_ref[...], preferred_element_type=jnp.float32) + b2_ref[...]
    m = _silu(z)
    u = _silu(jnp.dot(m, cw1_ref[...], preferred_element_type=jnp.float32) + cb1_ref[...])
    phi = jnp.dot(u, cw2_ref[...], preferred_element_type=jnp.float32)
    m_ref[...] = m
    phi_ref[...] = phi


def _edge_call(pre01, radial, w1c, b1, w2, b2, cw1, cb1, cw2, block_e=1000):
    E, D = pre01.shape
    grid = (E // block_e,)
    full = lambda shape: pl.BlockSpec(shape, lambda i: (0, 0))
    return pl.pallas_call(
        _edge_body,
        grid=grid,
        in_specs=[
            pl.BlockSpec((block_e, D), lambda i: (i, 0)),
            pl.BlockSpec((block_e, 1), lambda i: (i, 0)),
            full((1, D)), full((1, D)), full((D, D)), full((1, D)),
            full((D, D)), full((1, D)), full((D, 1)),
        ],
        out_specs=[
            pl.BlockSpec((block_e, D), lambda i: (i, 0)),
            pl.BlockSpec((block_e, 1), lambda i: (i, 0)),
        ],
        out_shape=[
            jax.ShapeDtypeStruct((E, D), jnp.float32),
            jax.ShapeDtypeStruct((E, 1), jnp.float32),
        ],
    )(pre01, radial, w1c, b1, w2, b2, cw1, cb1, cw2)


def _node_body_pq(h_ref, agg_ref, nw1a_ref, nw1b_ref, nb1_ref, nw2_ref, nb2_ref,
                  pwa_ref, pwb_ref, h_out, p_out, q_out):
    t = (jnp.dot(h_ref[...], nw1a_ref[...], preferred_element_type=jnp.float32)
         + jnp.dot(agg_ref[...], nw1b_ref[...], preferred_element_type=jnp.float32)
         + nb1_ref[...])
    out = jnp.dot(_silu(t), nw2_ref[...], preferred_element_type=jnp.float32) + nb2_ref[...]
    hn = h_ref[...] + out
    h_out[...] = hn
    p_out[...] = jnp.dot(hn, pwa_ref[...], preferred_element_type=jnp.float32)
    q_out[...] = jnp.dot(hn, pwb_ref[...], preferred_element_type=jnp.float32)


def _node_body_emb(h_ref, agg_ref, nw1a_ref, nw1b_ref, nb1_ref, nw2_ref, nb2_ref,
                   ew_ref, eb_ref, h_out):
    t = (jnp.dot(h_ref[...], nw1a_ref[...], preferred_element_type=jnp.float32)
         + jnp.dot(agg_ref[...], nw1b_ref[...], preferred_element_type=jnp.float32)
         + nb1_ref[...])
    out = jnp.dot(_silu(t), nw2_ref[...], preferred_element_type=jnp.float32) + nb2_ref[...]
    hn = h_ref[...] + out
    h_out[...] = jnp.dot(hn, ew_ref[...], preferred_element_type=jnp.float32) + eb_ref[...]


def _node_call(h, agg, nw1a, nw1b, nb1, nw2, nb2, wa, wb, last, block_n=1000):
    N, D = h.shape
    grid = (N // block_n,)
    full2 = lambda shape: pl.BlockSpec(shape, lambda i: (0, 0))
    blk = pl.BlockSpec((block_n, D), lambda i: (i, 0))
    if last:
        return pl.pallas_call(
            _node_body_emb,
            grid=grid,
            in_specs=[blk, blk, full2((D, D)), full2((D, D)), full2((1, D)),
                      full2((D, D)), full2((1, D)), full2((D, D)), full2((1, D))],
            out_specs=blk,
            out_shape=jax.ShapeDtypeStruct((N, D), jnp.float32),
        )(h, agg, nw1a, nw1b, nb1, nw2, nb2, wa, wb)
    return pl.pallas_call(
        _node_body_pq,
        grid=grid,
        in_specs=[blk, blk, full2((D, D)), full2((D, D)), full2((1, D)),
                  full2((D, D)), full2((1, D)), full2((D, D)), full2((D, D))],
        out_specs=[blk, blk, blk],
        out_shape=[jax.ShapeDtypeStruct((N, D), jnp.float32)] * 3,
    )(h, agg, nw1a, nw1b, nb1, nw2, nb2, wa, wb)


def _init_body(h_ref, ew_ref, eb_ref, pwa_ref, pwb_ref, h_out, p_out, q_out):
    hn = jnp.dot(h_ref[...], ew_ref[...], preferred_element_type=jnp.float32) + eb_ref[...]
    h_out[...] = hn
    p_out[...] = jnp.dot(hn, pwa_ref[...], preferred_element_type=jnp.float32)
    q_out[...] = jnp.dot(hn, pwb_ref[...], preferred_element_type=jnp.float32)


def _init_call(h, ew, eb, pwa, pwb, block_n=1000):
    N, D = h.shape
    grid = (N // block_n,)
    full2 = lambda shape: pl.BlockSpec(shape, lambda i: (0, 0))
    blk = pl.BlockSpec((block_n, D), lambda i: (i, 0))
    return pl.pallas_call(
        _init_body,
        grid=grid,
        in_specs=[blk, full2((D, D)), full2((1, D)), full2((D, D)), full2((D, D))],
        out_specs=[blk, blk, blk],
        out_shape=[jax.ShapeDtypeStruct((N, D), jnp.float32)] * 3,
    )(h, ew, eb, pwa, pwb)


def kernel(h, x, edge_index, emb_in_w, emb_in_b, edge_w1, edge_b1, edge_w2,
           edge_b2, node_w1, node_b1, node_w2, node_b2, coord_w1, coord_b1,
           coord_w2, emb_out_w, emb_out_b):
    N, D = h.shape
    L = edge_w1.shape[0]
    row = edge_index[0]
    col = edge_index[1]

    h1, P, Q = _init_call(h, emb_in_w, emb_in_b.reshape(1, D),
                          edge_w1[0, :D, :], edge_w1[0, D:2 * D, :])
    h = h1
    for l in range(L):
        cdiff = x[row] - x[col]
        radial = jnp.sum(cdiff * cdiff, axis=1, keepdims=True)
        pre01 = P[row] + Q[col]
        m, phi = _edge_call(pre01, radial,
                            edge_w1[l, 2 * D:2 * D + 1, :], edge_b1[l].reshape(1, D),
                            edge_w2[l], edge_b2[l].reshape(1, D),
                            coord_w1[l], coord_b1[l].reshape(1, D), coord_w2[l])
        trans = cdiff * phi
        s = jax.ops.segment_sum(trans, row, num_segments=N)
        c = jax.ops.segment_sum(jnp.ones((row.shape[0], 1), jnp.float32), row,
                                num_segments=N)
        x = x + s / jnp.maximum(c, 1.0)
        agg = jax.ops.segment_sum(m, row, num_segments=N)
        last = (l == L - 1)
        if last:
            h = _node_call(h, agg, node_w1[l, :D, :], node_w1[l, D:, :],
                           node_b1[l].reshape(1, D), node_w2[l],
                           node_b2[l].reshape(1, D), emb_out_w,
                           emb_out_b.reshape(1, D), last=True)
        else:
            h, P, Q = _node_call(h, agg, node_w1[l, :D, :], node_w1[l, D:, :],
                                 node_b1[l].reshape(1, D), node_w2[l],
                                 node_b2[l].reshape(1, D),
                                 edge_w1[l + 1, :D, :], edge_w1[l + 1, D:2 * D, :],
                                 last=False)
    return (h, x)
